# Initial kernel scaffold; baseline (speedup 1.0000x reference)
import functools

import jax
import jax.numpy as jnp
from jax import lax
from jax.experimental import pallas as pl
from jax.experimental.pallas import tpu as pltpu
from jax.experimental.pallas import tpu_sc as plsc

N_NODES_C = 10000
N_EDGES_C = 320000
D_IN = 128
D_PAD = 144
NC = 2
NS = 16
NHALF = N_NODES_C // NC
ACC_ROWS = NHALF + 8
EDGES_PER_TILE = N_EDGES_C // NS
BLK = 80
NBLK = EDGES_PER_TILE // BLK


def _sc_body(xpad_hbm, src_hbm, dst_hbm, u_hbm, zeros_hbm,
             accx_out, accux_out,
             src_v, dstl_v, u_v, rows_v, ux_v, sem,
             accx_sh, accux_sh):
    c = lax.axis_index("c")
    s = lax.axis_index("s")
    slice_rows = ACC_ROWS // NS
    r0 = s * slice_rows
    pltpu.sync_copy(zeros_hbm.at[pl.ds(r0, slice_rows)],
                    accx_sh.at[pl.ds(r0, slice_rows)])
    pltpu.sync_copy(zeros_hbm.at[pl.ds(r0, slice_rows)],
                    accux_sh.at[pl.ds(r0, slice_rows)])
    plsc.subcore_barrier()

    base_e = s * EDGES_PER_TILE
    half_lo = c * NHALF

    def block(i, carry):
        e0 = pl.multiple_of(base_e + i * BLK, BLK)
        pltpu.sync_copy(src_hbm.at[pl.ds(e0, BLK)], src_v)
        pltpu.sync_copy(dst_hbm.at[pl.ds(e0, BLK)], dstl_v)
        pltpu.sync_copy(u_hbm.at[pl.ds(e0, BLK)], u_v)
        pltpu.async_copy(xpad_hbm.at[src_v], rows_v, sem).wait()
        for g in range(BLK // 16):
            d = dstl_v[pl.ds(g * 16, 16)]
            dl = d - half_lo
            oob = (dl < 0) | (dl >= NHALF)
            dstl_v[pl.ds(g * 16, 16)] = jnp.where(oob, NHALF, dl)
            uraw = u_v[pl.ds(g * 16, 16)]
            u_v[pl.ds(g * 16, 16)] = jnp.clip(uraw, 0.0, 1.0)
        for j in range(BLK):
            idx = jnp.full((16,), j, dtype=jnp.int32)
            uj = plsc.load_gather(u_v, [idx])
            for k in range(D_PAD // 16):
                ux_v[j, pl.ds(k * 16, 16)] = rows_v[j, pl.ds(k * 16, 16)] * uj
        pltpu.sync_copy(rows_v, accx_sh.at[dstl_v], add=True)
        pltpu.sync_copy(ux_v, accux_sh.at[dstl_v], add=True)
        return carry

    lax.fori_loop(0, NBLK, block, 0)
    plsc.subcore_barrier()
    pltpu.sync_copy(accx_sh.at[pl.ds(r0, slice_rows)],
                    accx_out.at[c, pl.ds(r0, slice_rows)])
    pltpu.sync_copy(accux_sh.at[pl.ds(r0, slice_rows)],
                    accux_out.at[c, pl.ds(r0, slice_rows)])


@jax.jit
def _sc_aggregate(xpad, src, dst, u, zeros):
    mesh = plsc.VectorSubcoreMesh(core_axis_name="c", subcore_axis_name="s")
    f = pl.kernel(
        _sc_body,
        out_type=[
            jax.ShapeDtypeStruct((NC, ACC_ROWS, D_PAD), jnp.float32),
            jax.ShapeDtypeStruct((NC, ACC_ROWS, D_PAD), jnp.float32),
        ],
        mesh=mesh,
        scratch_types=[
            pltpu.VMEM((BLK,), jnp.int32),
            pltpu.VMEM((BLK,), jnp.int32),
            pltpu.VMEM((BLK,), jnp.float32),
            pltpu.VMEM((BLK, D_PAD), jnp.float32),
            pltpu.VMEM((BLK, D_PAD), jnp.float32),
            pltpu.SemaphoreType.DMA,
            pltpu.VMEM_SHARED((ACC_ROWS, D_PAD), jnp.float32),
            pltpu.VMEM_SHARED((ACC_ROWS, D_PAD), jnp.float32),
        ],
    )
    return f(xpad, src, dst, u, zeros)


def _tc_body(accx_ref, accux_ref, x_ref, w0_ref, w1m0_ref, rw_ref, o_ref):
    a = accx_ref[...]
    b = accux_ref[...]
    sx = a[:, :D_IN]
    deg = a[:, D_IN:D_IN + 1]
    su = b[:, :D_IN]
    m = jnp.dot(sx, w0_ref[...], preferred_element_type=jnp.float32,
                precision=lax.Precision.HIGHEST)
    m += jnp.dot(su, w1m0_ref[...], preferred_element_type=jnp.float32,
                 precision=lax.Precision.HIGHEST)
    root = jnp.dot(x_ref[...], rw_ref[...], preferred_element_type=jnp.float32,
                   precision=lax.Precision.HIGHEST)
    o_ref[...] = m / jnp.maximum(deg, 1.0) + root


@jax.jit
def _tc_combine(accx, accux, x, w0, w1m0, root_w):
    rows = 1000
    grid = N_NODES_C // rows
    return pl.pallas_call(
        _tc_body,
        grid=(grid,),
        in_specs=[
            pl.BlockSpec((rows, D_PAD), lambda i: (i, 0)),
            pl.BlockSpec((rows, D_PAD), lambda i: (i, 0)),
            pl.BlockSpec((rows, D_IN), lambda i: (i, 0)),
            pl.BlockSpec((D_IN, D_IN), lambda i: (0, 0)),
            pl.BlockSpec((D_IN, D_IN), lambda i: (0, 0)),
            pl.BlockSpec((D_IN, D_IN), lambda i: (0, 0)),
        ],
        out_specs=pl.BlockSpec((rows, D_IN), lambda i: (i, 0)),
        out_shape=jax.ShapeDtypeStruct((N_NODES_C, D_IN), jnp.float32),
    )(accx, accux, x, w0, w1m0, root_w)


def kernel(node_feature, edge_index, edge_feature, W, root_W):
    src = edge_index[0].astype(jnp.int32)
    dst = edge_index[1].astype(jnp.int32)
    u = edge_feature[:, 0]
    pad = jnp.zeros((N_NODES_C, D_PAD - D_IN), jnp.float32).at[:, 0].set(1.0)
    xpad = jnp.concatenate([node_feature, pad], axis=1)
    zeros = jnp.zeros((ACC_ROWS, D_PAD), jnp.float32)
    accx, accux = _sc_aggregate(xpad, src, dst, u, zeros)
    accx_full = accx[:, :NHALF, :].reshape(N_NODES_C, D_PAD)
    accux_full = accux[:, :NHALF, :].reshape(N_NODES_C, D_PAD)
    return _tc_combine(accx_full, accux_full, node_feature,
                       W[0], W[1] - W[0], root_W)


# speedup vs baseline: 2.1784x; 2.1784x over previous
import jax
import jax.numpy as jnp
from jax import lax
from jax.experimental import pallas as pl
from jax.experimental.pallas import tpu as pltpu
from jax.experimental.pallas import tpu_sc as plsc

N_NODES_C = 10000
N_EDGES_C = 320000
D_IN = 128
NC = 2
NS = 16
NHALF = N_NODES_C // NC
ACC_ROWS = 5120
EDGES_PER_TILE = N_EDGES_C // NS
BLK = 80
NBLK = EDGES_PER_TILE // BLK


def _splat_lane(vec, lane):
    idx = jnp.full((16, 1), lane, dtype=jnp.int32)
    dn = lax.GatherDimensionNumbers(
        offset_dims=(), collapsed_slice_dims=(0,), start_index_map=(0,))
    return lax.gather(vec, idx, dn, slice_sizes=(1,),
                      mode=lax.GatherScatterMode.PROMISE_IN_BOUNDS)


def _localize(dstl_v, half_lo, g):
    d = dstl_v[pl.ds(g * 16, 16)]
    dl = d - half_lo
    oob = (dl < 0) | (dl >= NHALF)
    dstl_v[pl.ds(g * 16, 16)] = jnp.where(oob, NHALF, dl)


def _sc_body(x_hbm, src_hbm, dst_hbm, u_hbm, zeros_hbm,
             accx_out, accux_out,
             src_v, dstl_v, u_v, rows_v, ux_v, sem,
             accx_sh, accux_sh):
    c = lax.axis_index("c")
    s = lax.axis_index("s")
    slice_rows = ACC_ROWS // NS
    r0 = s * slice_rows
    pltpu.sync_copy(zeros_hbm.at[pl.ds(r0, slice_rows)],
                    accx_sh.at[pl.ds(r0, slice_rows)])
    pltpu.sync_copy(zeros_hbm.at[pl.ds(r0, slice_rows)],
                    accux_sh.at[pl.ds(r0, slice_rows)])
    plsc.subcore_barrier()

    base_e = s * EDGES_PER_TILE
    half_lo = c * NHALF

    def block(i, carry):
        e0 = pl.multiple_of(base_e + i * BLK, BLK)
        pltpu.sync_copy(src_hbm.at[pl.ds(e0, BLK)], src_v)
        pltpu.sync_copy(dst_hbm.at[pl.ds(e0, BLK)], dstl_v)
        pltpu.sync_copy(u_hbm.at[pl.ds(e0, BLK)], u_v)
        pltpu.async_copy(x_hbm.at[src_v], rows_v, sem).wait()
        for g in range(BLK // 16):
            _localize(dstl_v, half_lo, g)
            uraw = u_v[pl.ds(g * 16, 16)]
            u_v[pl.ds(g * 16, 16)] = jnp.clip(uraw, 0.0, 1.0)
        for g in range(BLK // 16):
            uv = u_v[pl.ds(g * 16, 16)]
            for l in range(16):
                j = g * 16 + l
                uj = _splat_lane(uv, l)
                for k in range(D_IN // 16):
                    ux_v[j, pl.ds(k * 16, 16)] = (
                        rows_v[j, pl.ds(k * 16, 16)] * uj)
        pltpu.sync_copy(rows_v, accx_sh.at[dstl_v], add=True)
        pltpu.sync_copy(ux_v, accux_sh.at[dstl_v], add=True)
        return carry

    lax.fori_loop(0, NBLK, block, 0)
    plsc.subcore_barrier()
    pltpu.sync_copy(accx_sh.at[pl.ds(r0, slice_rows)],
                    accx_out.at[c, pl.ds(r0, slice_rows)])
    pltpu.sync_copy(accux_sh.at[pl.ds(r0, slice_rows)],
                    accux_out.at[c, pl.ds(r0, slice_rows)])


def _deg_body(dst_hbm, u_hbm, zeros_hbm, ones_hbm,
              deg_out,
              dstl_v, ones_v, deg_sh):
    c = lax.axis_index("c")
    s = lax.axis_index("s")
    slice_rows = ACC_ROWS // NS
    r0 = s * slice_rows
    pltpu.sync_copy(ones_hbm, ones_v)
    pltpu.sync_copy(zeros_hbm.at[pl.ds(r0, slice_rows)],
                    deg_sh.at[pl.ds(r0, slice_rows)])
    plsc.subcore_barrier()

    base_e = s * EDGES_PER_TILE
    half_lo = c * NHALF

    def block(i, carry):
        e0 = pl.multiple_of(base_e + i * BLK, BLK)
        pltpu.sync_copy(dst_hbm.at[pl.ds(e0, BLK)], dstl_v)
        for g in range(BLK // 16):
            _localize(dstl_v, half_lo, g)
        pltpu.sync_copy(ones_v, deg_sh.at[dstl_v], add=True)
        return carry

    lax.fori_loop(0, NBLK, block, 0)
    plsc.subcore_barrier()
    pltpu.sync_copy(deg_sh.at[pl.ds(r0, slice_rows)],
                    deg_out.at[c, pl.ds(r0, slice_rows)])


@jax.jit
def _sc_aggregate(x, src, dst, u, zeros, ones):
    mesh = plsc.VectorSubcoreMesh(core_axis_name="c", subcore_axis_name="s")
    acc_t = jax.ShapeDtypeStruct((NC, ACC_ROWS, D_IN), jnp.float32)
    main = pl.kernel(
        _sc_body,
        out_type=[acc_t, acc_t],
        mesh=mesh,
        scratch_types=[
            pltpu.VMEM((BLK,), jnp.int32),
            pltpu.VMEM((BLK,), jnp.int32),
            pltpu.VMEM((BLK,), jnp.float32),
            pltpu.VMEM((BLK, D_IN), jnp.float32),
            pltpu.VMEM((BLK, D_IN), jnp.float32),
            pltpu.SemaphoreType.DMA,
            pltpu.VMEM_SHARED((ACC_ROWS, D_IN), jnp.float32),
            pltpu.VMEM_SHARED((ACC_ROWS, D_IN), jnp.float32),
        ],
    )
    accx, accux = main(x, src, dst, u, zeros)
    degk = pl.kernel(
        _deg_body,
        out_type=[acc_t],
        mesh=mesh,
        scratch_types=[
            pltpu.VMEM((BLK,), jnp.int32),
            pltpu.VMEM((BLK, D_IN), jnp.float32),
            pltpu.VMEM_SHARED((ACC_ROWS, D_IN), jnp.float32),
        ],
    )
    (deg,) = degk(dst, u, zeros, ones)
    return accx, accux, deg


def _tc_body(sx_ref, sux_ref, deg_ref, x_ref, w0_ref, w1m0_ref, rw_ref,
             o_ref):
    sx = sx_ref[...]
    su = sux_ref[...]
    deg = deg_ref[:, 0:1]
    m = jnp.dot(sx, w0_ref[...], preferred_element_type=jnp.float32,
                precision=lax.Precision.HIGHEST)
    m += jnp.dot(su, w1m0_ref[...], preferred_element_type=jnp.float32,
                 precision=lax.Precision.HIGHEST)
    root = jnp.dot(x_ref[...], rw_ref[...], preferred_element_type=jnp.float32,
                   precision=lax.Precision.HIGHEST)
    o_ref[...] = m / jnp.maximum(deg, 1.0) + root


@jax.jit
def _tc_combine(sx, sux, deg, x, w0, w1m0, root_w):
    rows = 1000
    grid = N_NODES_C // rows
    return pl.pallas_call(
        _tc_body,
        grid=(grid,),
        in_specs=[
            pl.BlockSpec((rows, D_IN), lambda i: (i, 0)),
            pl.BlockSpec((rows, D_IN), lambda i: (i, 0)),
            pl.BlockSpec((rows, D_IN), lambda i: (i, 0)),
            pl.BlockSpec((rows, D_IN), lambda i: (i, 0)),
            pl.BlockSpec((D_IN, D_IN), lambda i: (0, 0)),
            pl.BlockSpec((D_IN, D_IN), lambda i: (0, 0)),
            pl.BlockSpec((D_IN, D_IN), lambda i: (0, 0)),
        ],
        out_specs=pl.BlockSpec((rows, D_IN), lambda i: (i, 0)),
        out_shape=jax.ShapeDtypeStruct((N_NODES_C, D_IN), jnp.float32),
    )(sx, sux, deg, x, w0, w1m0, root_w)


def kernel(node_feature, edge_index, edge_feature, W, root_W):
    src = edge_index[0].astype(jnp.int32)
    dst = edge_index[1].astype(jnp.int32)
    u = edge_feature[:, 0]
    zeros = jnp.zeros((ACC_ROWS, D_IN), jnp.float32)
    ones = jnp.ones((BLK, D_IN), jnp.float32)
    accx, accux, accdeg = _sc_aggregate(node_feature, src, dst, u, zeros,
                                        ones)
    sx = accx[:, :NHALF, :].reshape(N_NODES_C, D_IN)
    sux = accux[:, :NHALF, :].reshape(N_NODES_C, D_IN)
    deg = accdeg[:, :NHALF, :].reshape(N_NODES_C, D_IN)
    return _tc_combine(sx, sux, deg, node_feature,
                       W[0], W[1] - W[0], root_W)
